# Initial kernel scaffold; baseline (speedup 1.0000x reference)
import jax
import jax.numpy as jnp
from jax.experimental import pallas as pl


def kernel(teacher_feats, student_feats, ref_perm, shared_perm):
    raise NotImplementedError("write your pallas kernel here")


# speedup vs baseline: 6.8795x; 6.8795x over previous
import functools

import jax
import jax.numpy as jnp
from jax import lax
from jax.experimental import pallas as pl
from jax.experimental.pallas import tpu as pltpu
from jax.experimental.pallas import tpu_sc as plsc

_B, _ST, _SS, _P, _D = 2, 8, 4, 1369, 1024
_R = 128
_S = 128
_K = 4
_NE = 4
_EXTRA_STRIDE = 2
_SHT = (2, 4, 6)
_SHS = (1, 2, 3)
_EPS = 1e-8
_RK = _R * _K
_NCAND = _NE * _K


def _sc_perm_gather(t_flat, s_flat, idx_t, idx_s):
    info = plsc.get_sparse_core_info()
    nw = info.num_cores * info.num_subcores
    n_rows = idx_t.shape[0]
    per_w = n_rows // nw
    mesh = plsc.VectorSubcoreMesh(core_axis_name="c", subcore_axis_name="s")

    @functools.partial(
        pl.kernel,
        out_type=(
            jax.ShapeDtypeStruct((n_rows, _D), jnp.float32),
            jax.ShapeDtypeStruct((n_rows, _D), jnp.float32),
        ),
        mesh=mesh,
        scratch_types=[
            pltpu.VMEM((per_w,), jnp.int32),
            pltpu.VMEM((per_w, _D), jnp.float32),
            pltpu.SemaphoreType.DMA,
        ],
    )
    def k(t_hbm, s_hbm, it_hbm, is_hbm, ot_hbm, os_hbm, idx_v, rows_v, sem):
        wid = lax.axis_index("s") * info.num_cores + lax.axis_index("c")
        base = wid * per_w
        pltpu.sync_copy(it_hbm.at[pl.ds(base, per_w)], idx_v)
        pltpu.async_copy(t_hbm.at[idx_v], rows_v, sem).wait()
        pltpu.sync_copy(rows_v, ot_hbm.at[pl.ds(base, per_w)])
        pltpu.sync_copy(is_hbm.at[pl.ds(base, per_w)], idx_v)
        pltpu.async_copy(s_hbm.at[idx_v], rows_v, sem).wait()
        pltpu.sync_copy(rows_v, os_hbm.at[pl.ds(base, per_w)])

    return k(t_flat, s_flat, idx_t, idx_s)


def _simtopk_body(t_ref, r_ref, out_i_ref, scr_v, scr_i):
    b = pl.program_id(0)
    e = pl.program_id(1)

    @pl.when(e == 0)
    def _init():
        scr_v[...] = jnp.full((_R, _NCAND), -jnp.inf, jnp.float32)
        scr_i[...] = jnp.zeros((_R, _NCAND), jnp.int32)

    f = t_ref[0, 0]
    fn = jnp.maximum(jnp.sqrt(jnp.sum(f * f, axis=-1, keepdims=True)), 1e-12)
    etn = f / fn
    r = r_ref[0]
    rn = jnp.maximum(jnp.sqrt(jnp.sum(r * r, axis=-1, keepdims=True)), 1e-12)
    rtn = r / rn
    sim = lax.dot_general(rtn, etn, (((1,), (1,)), ((), ())),
                          preferred_element_type=jnp.float32)

    iota = lax.broadcasted_iota(jnp.int32, (_R, _P), 1)
    lane = lax.broadcasted_iota(jnp.int32, (_R, _NCAND), 1)
    base = (b * _ST + _EXTRA_STRIDE * e + 1) * _P
    sv = scr_v[...]
    si = scr_i[...]
    for j in range(_K):
        m = jnp.max(sim, axis=1, keepdims=True)
        pos = jnp.min(jnp.where(sim == m, iota, jnp.int32(2 ** 30)),
                      axis=1, keepdims=True)
        sim = jnp.where(iota == pos, -jnp.inf, sim)
        slot = e * _K + j
        sv = jnp.where(lane == slot, m, sv)
        si = jnp.where(lane == slot, pos + base, si)
    scr_v[...] = sv
    scr_i[...] = si

    @pl.when(e == _NE - 1)
    def _emit():
        v = scr_v[...]
        ci = scr_i[...]
        lane4 = lax.broadcasted_iota(jnp.int32, (_R, _K), 1)
        res = jnp.zeros((_R, _K), jnp.int32)
        for j in range(_K):
            m = jnp.max(v, axis=1, keepdims=True)
            pos = jnp.min(jnp.where(v == m, lane, jnp.int32(2 ** 30)),
                          axis=1, keepdims=True)
            sel = jnp.sum(jnp.where(lane == pos, ci, 0), axis=1, keepdims=True)
            res = jnp.where(lane4 == j, sel, res)
            v = jnp.where(lane == pos, -jnp.inf, v)
        out_i_ref[0] = res


def _tc_sim_topk(teacher_feats, ref_t):
    return pl.pallas_call(
        _simtopk_body,
        grid=(_B, _NE),
        in_specs=[
            pl.BlockSpec((1, 1, _P, _D), lambda b, e: (b, _EXTRA_STRIDE * e + 1, 0, 0)),
            pl.BlockSpec((1, _R, _D), lambda b, e: (b, 0, 0)),
        ],
        out_specs=pl.BlockSpec((1, _R, _K), lambda b, e: (b, 0, 0)),
        out_shape=jax.ShapeDtypeStruct((_B, _R, _K), jnp.int32),
        scratch_shapes=[
            pltpu.VMEM((_R, _NCAND), jnp.float32),
            pltpu.VMEM((_R, _NCAND), jnp.int32),
        ],
        compiler_params=pltpu.CompilerParams(
            dimension_semantics=("arbitrary", "arbitrary")),
    )(teacher_feats, ref_t)


def _sc_topk_gather(gidx, t_flat):
    info = plsc.get_sparse_core_info()
    nw = info.num_cores * info.num_subcores
    n_rows = gidx.shape[0]
    per_w = n_rows // nw
    mesh = plsc.VectorSubcoreMesh(core_axis_name="c", subcore_axis_name="s")

    @functools.partial(
        pl.kernel,
        out_type=jax.ShapeDtypeStruct((n_rows, _D), jnp.float32),
        mesh=mesh,
        scratch_types=[
            pltpu.VMEM((per_w,), jnp.int32),
            pltpu.VMEM((per_w, _D), jnp.float32),
            pltpu.SemaphoreType.DMA,
        ],
    )
    def k(i_hbm, t_hbm, out_hbm, idx_v, rows_v, sem):
        wid = lax.axis_index("s") * info.num_cores + lax.axis_index("c")
        base = wid * per_w
        pltpu.sync_copy(i_hbm.at[pl.ds(base, per_w)], idx_v)
        pltpu.async_copy(t_hbm.at[idx_v], rows_v, sem).wait()
        pltpu.sync_copy(rows_v, out_hbm.at[pl.ds(base, per_w)])

    return k(gidx, t_flat)


def _huber(pred, target):
    e = pred - target
    ae = jnp.abs(e)
    return jnp.where(ae <= 1.0, 0.5 * e * e, ae - 0.5)


def _angles_body(rt_ref, rs_ref, sht_ref, shs_ref, h_ref, out_ref):
    cd = (((1,), (1,)), ((), ()))
    md = (((1,), (0,)), ((), ()))
    hi = lax.Precision.HIGHEST
    row = lax.broadcasted_iota(jnp.int32, (_RK, _R), 0)
    col = lax.broadcasted_iota(jnp.int32, (_RK, _R), 1)
    E = (row // _K == col).astype(jnp.float32)
    ones = jnp.ones((1, _D), jnp.float32)

    acc = jnp.float32(0.0)
    for b in range(_B):
        H = h_ref[b]
        hh = jnp.sum(H * H, axis=-1, keepdims=True)
        side = []
        for r_ref_ in (rt_ref, rs_ref):
            rep = lax.dot_general(E, r_ref_[b], md,
                                  preferred_element_type=jnp.float32,
                                  precision=hi)
            rh = jnp.sum(rep * H, axis=-1, keepdims=True)
            rr = jnp.sum(rep * rep, axis=-1, keepdims=True)
            side.append((rep, rh, rr))
        for i in range(len(_SHT)):
            angles = []
            for (sh_ref_, (rep, rh, rr)) in ((sht_ref, side[0]),
                                             (shs_ref, side[1])):
                sh = sh_ref_[i, b]
                rs2 = lax.dot_general(rep, sh, cd,
                                      preferred_element_type=jnp.float32,
                                      precision=hi)
                sh2 = lax.dot_general(H, sh, cd,
                                      preferred_element_type=jnp.float32,
                                      precision=hi)
                ss = lax.dot_general(ones, sh * sh, cd,
                                     preferred_element_type=jnp.float32,
                                     precision=hi)
                dot1 = sh2 - rs2 - rh + rr
                na1 = jnp.maximum(jnp.sqrt(jnp.maximum(ss + rr - 2.0 * rs2, 0.0)), _EPS)
                nb1 = jnp.maximum(jnp.sqrt(jnp.maximum(hh + rr - 2.0 * rh, 0.0)), _EPS)
                a1 = dot1 / (na1 * nb1)
                dot2 = rs2 - sh2 - rh + hh
                na2 = jnp.maximum(jnp.sqrt(jnp.maximum(rr + hh - 2.0 * rh, 0.0)), _EPS)
                nb2 = jnp.maximum(jnp.sqrt(jnp.maximum(ss + hh - 2.0 * sh2, 1e-12)), _EPS)
                a2 = dot2 / (na2 * nb2)
                dot3 = rh - sh2 - rs2 + ss
                na3 = jnp.maximum(jnp.sqrt(jnp.maximum(rr + ss - 2.0 * rs2, 0.0)), _EPS)
                nb3 = jnp.maximum(jnp.sqrt(jnp.maximum(hh + ss - 2.0 * sh2, 1e-12)), _EPS)
                a3 = dot3 / (na3 * nb3)
                angles.append((a1, a2, a3))
            (t1, t2, t3), (s1, s2, s3) = angles
            acc = acc + jnp.sum(_huber(s1, t1)) + jnp.sum(_huber(s2, t2)) \
                      + jnp.sum(_huber(s3, t3))
    total = float(len(_SHT) * _B * _R * _S * _K)
    out_ref[...] = jnp.reshape(acc / total, (1, 1))


def _tc_angles(ref_t, ref_s, shared_t, shared_s, h):
    return pl.pallas_call(
        _angles_body,
        out_shape=jax.ShapeDtypeStruct((1, 1), jnp.float32),
    )(ref_t, ref_s, shared_t, shared_s, h)


def kernel(teacher_feats, student_feats, ref_perm, shared_perm):
    t_flat = teacher_feats.reshape(_B * _ST * _P, _D)
    s_flat = student_feats.reshape(_B * _SS * _P, _D)

    boff_t = jnp.arange(_B, dtype=jnp.int32) * (_ST * _P)
    boff_s = jnp.arange(_B, dtype=jnp.int32) * (_SS * _P)
    idx_t = jnp.concatenate([
        (boff_t[:, None] + ref_perm[None, :]).reshape(-1),
        (boff_t[None, :, None] + jnp.array(_SHT, jnp.int32)[:, None, None] * _P
         + shared_perm[None, None, :]).reshape(-1),
    ])
    idx_s = jnp.concatenate([
        (boff_s[:, None] + ref_perm[None, :]).reshape(-1),
        (boff_s[None, :, None] + jnp.array(_SHS, jnp.int32)[:, None, None] * _P
         + shared_perm[None, None, :]).reshape(-1),
    ])

    out_t, out_s = _sc_perm_gather(t_flat, s_flat, idx_t, idx_s)
    ref_t = out_t[:_B * _R].reshape(_B, _R, _D)
    shared_t = out_t[_B * _R:].reshape(len(_SHT), _B, _S, _D)
    ref_s = out_s[:_B * _R].reshape(_B, _R, _D)
    shared_s = out_s[_B * _R:].reshape(len(_SHS), _B, _S, _D)

    gidx = _tc_sim_topk(teacher_feats, ref_t)
    h = _sc_topk_gather(gidx.reshape(_B * _R * _K), t_flat).reshape(_B, _RK, _D)

    out = _tc_angles(ref_t, ref_s, shared_t, shared_s, h)
    return out[0, 0]


# speedup vs baseline: 8.9478x; 1.3006x over previous
import functools

import jax
import jax.numpy as jnp
from jax import lax
from jax.experimental import pallas as pl
from jax.experimental.pallas import tpu as pltpu
from jax.experimental.pallas import tpu_sc as plsc

_B, _ST, _SS, _P, _D = 2, 8, 4, 1369, 1024
_R = 128
_S = 128
_K = 4
_NE = 4
_EXTRA_STRIDE = 2
_SHT = (2, 4, 6)
_SHS = (1, 2, 3)
_EPS = 1e-8
_RK = _R * _K
_PPAD = 1376
_NCAND = _NE * _K


def _sc_perm_gather(teacher_feats, student_feats, perm_tab):
    info = plsc.get_sparse_core_info()
    nw = info.num_cores * info.num_subcores
    n_rows = 8 * _R
    per_w = n_rows // nw
    mesh = plsc.VectorSubcoreMesh(core_axis_name="c", subcore_axis_name="s")

    @functools.partial(
        pl.kernel,
        out_type=(
            jax.ShapeDtypeStruct((n_rows, _D), jnp.float32),
            jax.ShapeDtypeStruct((n_rows, _D), jnp.float32),
        ),
        mesh=mesh,
        scratch_types=[
            pltpu.VMEM((per_w,), jnp.int32),
            pltpu.VMEM((per_w, _D), jnp.float32),
            pltpu.SemaphoreType.DMA,
        ],
    )
    def k(t_hbm, s_hbm, ptab_hbm, ot_hbm, os_hbm, idx_v, rows_v, sem):
        wid = lax.axis_index("s") * info.num_cores + lax.axis_index("c")
        g = wid // 4
        part = wid % 4
        is_ref = g < 2
        b = jnp.where(is_ref, g, (g - 2) % 2)
        i = (g - 2) // 2
        f_t = jnp.where(is_ref, 0, 2 + 2 * i)
        f_s = jnp.where(is_ref, 0, 1 + i)
        psel = jnp.where(is_ref, 0, 1)
        base = wid * per_w
        pltpu.sync_copy(ptab_hbm.at[psel, pl.ds(part * per_w, per_w)], idx_v)
        pltpu.async_copy(t_hbm.at[b, f_t].at[idx_v], rows_v, sem).wait()
        pltpu.sync_copy(rows_v, ot_hbm.at[pl.ds(base, per_w)])
        pltpu.async_copy(s_hbm.at[b, f_s].at[idx_v], rows_v, sem).wait()
        pltpu.sync_copy(rows_v, os_hbm.at[pl.ds(base, per_w)])

    return k(teacher_feats, student_feats, perm_tab)


def _simtopk_body(t_ref, r_ref, out_i_ref, lin_ref, scr_v, scr_i):
    b = pl.program_id(0)
    e = pl.program_id(1)

    @pl.when(e == 0)
    def _init():
        scr_v[...] = jnp.full((_R, _NCAND), -jnp.inf, jnp.float32)
        scr_i[...] = jnp.zeros((_R, _NCAND), jnp.int32)

    f = t_ref[0, 0]
    lin_ref[0, :_P, :] = f
    fn = jnp.maximum(jnp.sqrt(jnp.sum(f * f, axis=-1, keepdims=True)), 1e-12)
    etn = f / fn
    r = r_ref[0]
    rn = jnp.maximum(jnp.sqrt(jnp.sum(r * r, axis=-1, keepdims=True)), 1e-12)
    rtn = r / rn
    sim = lax.dot_general(rtn, etn, (((1,), (1,)), ((), ())),
                          preferred_element_type=jnp.float32)

    iota = lax.broadcasted_iota(jnp.int32, (_R, _P), 1)
    lane = lax.broadcasted_iota(jnp.int32, (_R, _NCAND), 1)
    base = e * _PPAD
    sv = scr_v[...]
    si = scr_i[...]
    for j in range(_K):
        m = jnp.max(sim, axis=1, keepdims=True)
        pos = jnp.min(jnp.where(sim == m, iota, jnp.int32(2 ** 30)),
                      axis=1, keepdims=True)
        sim = jnp.where(iota == pos, -jnp.inf, sim)
        slot = e * _K + j
        sv = jnp.where(lane == slot, m, sv)
        si = jnp.where(lane == slot, pos + base, si)
    scr_v[...] = sv
    scr_i[...] = si

    @pl.when(e == _NE - 1)
    def _emit():
        v = scr_v[...]
        ci = scr_i[...]
        lane4 = lax.broadcasted_iota(jnp.int32, (_R, _K), 1)
        res = jnp.zeros((_R, _K), jnp.int32)
        for j in range(_K):
            m = jnp.max(v, axis=1, keepdims=True)
            pos = jnp.min(jnp.where(v == m, lane, jnp.int32(2 ** 30)),
                          axis=1, keepdims=True)
            sel = jnp.sum(jnp.where(lane == pos, ci, 0), axis=1, keepdims=True)
            res = jnp.where(lane4 == j, sel, res)
            v = jnp.where(lane == pos, -jnp.inf, v)
        out_i_ref[0] = res


def _tc_sim_topk(teacher_feats, ref_t):
    return pl.pallas_call(
        _simtopk_body,
        grid=(_B, _NE),
        in_specs=[
            pl.BlockSpec((1, 1, _P, _D), lambda b, e: (b, _EXTRA_STRIDE * e + 1, 0, 0)),
            pl.BlockSpec((1, _R, _D), lambda b, e: (b, 0, 0)),
        ],
        out_specs=[
            pl.BlockSpec((1, _R, _K), lambda b, e: (b, 0, 0)),
            pl.BlockSpec((1, _PPAD, _D), lambda b, e: (b, e, 0)),
        ],
        out_shape=[
            jax.ShapeDtypeStruct((_B, _R, _K), jnp.int32),
            jax.ShapeDtypeStruct((_B, _NE * _PPAD, _D), jnp.float32),
        ],
        scratch_shapes=[
            pltpu.VMEM((_R, _NCAND), jnp.float32),
            pltpu.VMEM((_R, _NCAND), jnp.int32),
        ],
        compiler_params=pltpu.CompilerParams(
            dimension_semantics=("arbitrary", "arbitrary")),
    )(teacher_feats, ref_t)


def _sc_topk_gather(gidx, extra_lin):
    info = plsc.get_sparse_core_info()
    nw = info.num_cores * info.num_subcores
    n_rows = gidx.shape[0]
    per_w = n_rows // nw
    w_per_b = nw // _B
    mesh = plsc.VectorSubcoreMesh(core_axis_name="c", subcore_axis_name="s")

    @functools.partial(
        pl.kernel,
        out_type=jax.ShapeDtypeStruct((n_rows, _D), jnp.float32),
        mesh=mesh,
        scratch_types=[
            pltpu.VMEM((per_w,), jnp.int32),
            pltpu.VMEM((per_w, _D), jnp.float32),
            pltpu.SemaphoreType.DMA,
        ],
    )
    def k(i_hbm, t_hbm, out_hbm, idx_v, rows_v, sem):
        wid = lax.axis_index("s") * info.num_cores + lax.axis_index("c")
        base = wid * per_w
        b = wid // w_per_b
        pltpu.sync_copy(i_hbm.at[pl.ds(base, per_w)], idx_v)
        pltpu.async_copy(t_hbm.at[b].at[idx_v], rows_v, sem).wait()
        pltpu.sync_copy(rows_v, out_hbm.at[pl.ds(base, per_w)])

    return k(gidx, extra_lin)


def _huber(pred, target):
    e = pred - target
    ae = jnp.abs(e)
    return jnp.where(ae <= 1.0, 0.5 * e * e, ae - 0.5)


def _angles_body(rt_ref, rs_ref, sht_ref, shs_ref, h_ref, out_ref):
    cd = (((1,), (1,)), ((), ()))
    md = (((1,), (0,)), ((), ()))
    hi = lax.Precision.HIGHEST
    row = lax.broadcasted_iota(jnp.int32, (_RK, _R), 0)
    col = lax.broadcasted_iota(jnp.int32, (_RK, _R), 1)
    E = (row // _K == col).astype(jnp.float32)
    ones = jnp.ones((1, _D), jnp.float32)

    acc = jnp.float32(0.0)
    for b in range(_B):
        H = h_ref[b]
        hh = jnp.sum(H * H, axis=-1, keepdims=True)
        side = []
        for r_ref_ in (rt_ref, rs_ref):
            rep = lax.dot_general(E, r_ref_[b], md,
                                  preferred_element_type=jnp.float32,
                                  precision=hi)
            rh = jnp.sum(rep * H, axis=-1, keepdims=True)
            rr = jnp.sum(rep * rep, axis=-1, keepdims=True)
            side.append((rep, rh, rr))
        for i in range(len(_SHT)):
            angles = []
            for (sh_ref_, (rep, rh, rr)) in ((sht_ref, side[0]),
                                             (shs_ref, side[1])):
                sh = sh_ref_[i, b]
                rs2 = lax.dot_general(rep, sh, cd,
                                      preferred_element_type=jnp.float32,
                                      precision=hi)
                sh2 = lax.dot_general(H, sh, cd,
                                      preferred_element_type=jnp.float32,
                                      precision=hi)
                ss = lax.dot_general(ones, sh * sh, cd,
                                     preferred_element_type=jnp.float32,
                                     precision=hi)
                dot1 = sh2 - rs2 - rh + rr
                na1 = jnp.maximum(jnp.sqrt(jnp.maximum(ss + rr - 2.0 * rs2, 0.0)), _EPS)
                nb1 = jnp.maximum(jnp.sqrt(jnp.maximum(hh + rr - 2.0 * rh, 0.0)), _EPS)
                a1 = dot1 / (na1 * nb1)
                dot2 = rs2 - sh2 - rh + hh
                na2 = jnp.maximum(jnp.sqrt(jnp.maximum(rr + hh - 2.0 * rh, 0.0)), _EPS)
                nb2 = jnp.maximum(jnp.sqrt(jnp.maximum(ss + hh - 2.0 * sh2, 1e-12)), _EPS)
                a2 = dot2 / (na2 * nb2)
                dot3 = rh - sh2 - rs2 + ss
                na3 = jnp.maximum(jnp.sqrt(jnp.maximum(rr + ss - 2.0 * rs2, 0.0)), _EPS)
                nb3 = jnp.maximum(jnp.sqrt(jnp.maximum(hh + ss - 2.0 * sh2, 1e-12)), _EPS)
                a3 = dot3 / (na3 * nb3)
                angles.append((a1, a2, a3))
            (t1, t2, t3), (s1, s2, s3) = angles
            acc = acc + jnp.sum(_huber(s1, t1)) + jnp.sum(_huber(s2, t2)) \
                      + jnp.sum(_huber(s3, t3))
    total = float(len(_SHT) * _B * _R * _S * _K)
    out_ref[...] = jnp.reshape(acc / total, (1, 1))


def _tc_angles(ref_t, ref_s, shared_t, shared_s, h):
    return pl.pallas_call(
        _angles_body,
        out_shape=jax.ShapeDtypeStruct((1, 1), jnp.float32),
    )(ref_t, ref_s, shared_t, shared_s, h)


def kernel(teacher_feats, student_feats, ref_perm, shared_perm):
    perm_tab = jnp.stack([ref_perm, shared_perm])

    out_t, out_s = _sc_perm_gather(teacher_feats, student_feats, perm_tab)
    ref_t = out_t[:_B * _R].reshape(_B, _R, _D)
    shared_t = out_t[_B * _R:].reshape(len(_SHT), _B, _S, _D)
    ref_s = out_s[:_B * _R].reshape(_B, _R, _D)
    shared_s = out_s[_B * _R:].reshape(len(_SHS), _B, _S, _D)

    gidx, extra_lin = _tc_sim_topk(teacher_feats, ref_t)
    h = _sc_topk_gather(gidx.reshape(_B * _R * _K), extra_lin).reshape(_B, _RK, _D)

    out = _tc_angles(ref_t, ref_s, shared_t, shared_s, h)
    return out[0, 0]
